# baseline (speedup 1.0000x reference)
import functools

import jax
import jax.numpy as jnp
from jax import lax
from jax.experimental import pallas as pl
from jax.experimental.pallas import tpu as pltpu
from jax.experimental.pallas import tpu_sc as plsc

_B = 16384
_V = 1000000
_D = 64
_NEG = 5

_NC = 2
_NS = 16
_NW = _NC * _NS

_CB = _B // _NW
_NB = _B * _NEG // _NW
_CHUNK = 512
_NCHUNKS = (_CB + _CB + _NB) // _CHUNK
_NBUF = 3


def _body(center_w, context_w, cidx, xidx, nidx, out_c, out_x, out_n,
          idx_v, rows_v, gsem, ssem):
    wid = lax.axis_index("s") * _NC + lax.axis_index("c")

    pltpu.sync_copy(cidx.at[pl.ds(wid * _CB, _CB)], idx_v.at[0])
    pltpu.sync_copy(xidx.at[pl.ds(wid * _CB, _CB)], idx_v.at[1])
    for j in range(_NB // _CHUNK):
        pltpu.sync_copy(nidx.at[pl.ds(wid * _NB + j * _CHUNK, _CHUNK)],
                        idx_v.at[2 + j])

    base = wid * _CB
    nbase = wid * _NB
    chunks = [(center_w, 0, out_c, base), (context_w, 1, out_x, base)]
    chunks += [(context_w, 2 + j, out_n, nbase + j * _CHUNK)
               for j in range(_NB // _CHUNK)]

    gathers = [None] * _NCHUNKS
    scatters = [None] * _NCHUNKS
    for i, (table, row, out, off) in enumerate(chunks):
        if i >= _NBUF:
            scatters[i - _NBUF].wait()
        gathers[i] = pltpu.async_copy(
            table.at[idx_v.at[row]], rows_v.at[i % _NBUF], gsem)
        if i > 0:
            _, _, pout, poff = chunks[i - 1]
            gathers[i - 1].wait()
            scatters[i - 1] = pltpu.async_copy(
                rows_v.at[(i - 1) % _NBUF], pout.at[pl.ds(poff, _CHUNK)],
                ssem)
    gathers[_NCHUNKS - 1].wait()
    last_table, _, lout, loff = chunks[_NCHUNKS - 1]
    scatters[_NCHUNKS - 1] = pltpu.async_copy(
        rows_v.at[(_NCHUNKS - 1) % _NBUF],
        lout.at[pl.ds(loff, _CHUNK)], ssem)
    for i in range(max(0, _NCHUNKS - _NBUF), _NCHUNKS):
        scatters[i].wait()


@jax.jit
def _run(center_nodes, context_nodes, neg_flat, center_W, context_W):
    mesh = plsc.VectorSubcoreMesh(core_axis_name="c", subcore_axis_name="s")
    fn = functools.partial(
        pl.kernel,
        mesh=mesh,
        out_type=(
            jax.ShapeDtypeStruct((_B, _D), jnp.float32),
            jax.ShapeDtypeStruct((_B, _D), jnp.float32),
            jax.ShapeDtypeStruct((_B * _NEG, _D), jnp.float32),
        ),
        scratch_types=[
            pltpu.VMEM((_NCHUNKS, _CHUNK), jnp.int32),
            pltpu.VMEM((_NBUF, _CHUNK, _D), jnp.float32),
            pltpu.SemaphoreType.DMA,
            pltpu.SemaphoreType.DMA,
        ],
        compiler_params=pltpu.CompilerParams(use_tc_tiling_on_sc=False),
    )(_body)
    return fn(center_W, context_W, center_nodes, context_nodes, neg_flat)


def kernel(center_nodes, context_nodes, negative_nodes, center_W, context_W):
    neg_flat = negative_nodes.reshape(_B * _NEG)
    out_c, out_x, out_n = _run(
        center_nodes.astype(jnp.int32),
        context_nodes.astype(jnp.int32),
        neg_flat.astype(jnp.int32),
        center_W,
        context_W,
    )
    return out_c, out_x, out_n.reshape(_B, _NEG, _D)


# speedup vs baseline: 1.0107x; 1.0107x over previous
import functools

import jax
import jax.numpy as jnp
from jax import lax
from jax.experimental import pallas as pl
from jax.experimental.pallas import tpu as pltpu
from jax.experimental.pallas import tpu_sc as plsc

_B = 16384
_V = 1000000
_D = 64
_NEG = 5

_NC = 2
_NS = 16
_NW = _NC * _NS

_CB = _B // _NW
_NB = _B * _NEG // _NW
_CHUNK = 512
_NBUF = 3


def _gather_pipeline(table, schedule, idx_v, rows_v, gsem, ssem):
    n = len(schedule)
    for i, (src, soff, row, _, _) in enumerate(schedule):
        pltpu.sync_copy(src.at[pl.ds(soff, _CHUNK)], idx_v.at[row])
    gathers = [None] * n
    scatters = [None] * n
    for i, (_, _, row, out, off) in enumerate(schedule):
        if i >= _NBUF:
            scatters[i - _NBUF].wait()
        gathers[i] = pltpu.async_copy(
            table.at[idx_v.at[row]], rows_v.at[i % _NBUF], gsem)
        if i > 0:
            _, _, _, pout, poff = schedule[i - 1]
            gathers[i - 1].wait()
            scatters[i - 1] = pltpu.async_copy(
                rows_v.at[(i - 1) % _NBUF], pout.at[pl.ds(poff, _CHUNK)],
                ssem)
    gathers[n - 1].wait()
    _, _, _, lout, loff = schedule[n - 1]
    scatters[n - 1] = pltpu.async_copy(
        rows_v.at[(n - 1) % _NBUF], lout.at[pl.ds(loff, _CHUNK)], ssem)
    for i in range(max(0, n - _NBUF), n):
        scatters[i].wait()


def _center_body(center_w, cidx, out_c, idx_v, rows_v, gsem, ssem):
    wid = lax.axis_index("s") * _NC + lax.axis_index("c")
    base = wid * _CB
    _gather_pipeline(center_w,
                     [(cidx, base, 0, out_c, base)],
                     idx_v, rows_v, gsem, ssem)


def _context_body(context_w, xidx, nidx, out_x, out_n, idx_v, rows_v,
                  gsem, ssem):
    wid = lax.axis_index("s") * _NC + lax.axis_index("c")
    base = wid * _CB
    nbase = wid * _NB
    schedule = [(xidx, base, 0, out_x, base)]
    schedule += [(nidx, nbase + j * _CHUNK, 1 + j, out_n, nbase + j * _CHUNK)
                 for j in range(_NB // _CHUNK)]
    _gather_pipeline(context_w, schedule, idx_v, rows_v, gsem, ssem)


def _make_kernel(body, n_idx_rows, out_type):
    mesh = plsc.VectorSubcoreMesh(core_axis_name="c", subcore_axis_name="s")
    return functools.partial(
        pl.kernel,
        mesh=mesh,
        out_type=out_type,
        scratch_types=[
            pltpu.VMEM((n_idx_rows, _CHUNK), jnp.int32),
            pltpu.VMEM((_NBUF, _CHUNK, _D), jnp.float32),
            pltpu.SemaphoreType.DMA,
            pltpu.SemaphoreType.DMA,
        ],
        compiler_params=pltpu.CompilerParams(use_tc_tiling_on_sc=False),
    )(body)


@jax.jit
def _run(center_nodes, context_nodes, neg_flat, center_W, context_W):
    k_center = _make_kernel(
        _center_body, 1,
        jax.ShapeDtypeStruct((_B, _D), jnp.float32))
    k_context = _make_kernel(
        _context_body, 1 + _NB // _CHUNK,
        (jax.ShapeDtypeStruct((_B, _D), jnp.float32),
         jax.ShapeDtypeStruct((_B * _NEG, _D), jnp.float32)))
    out_c = k_center(center_W, center_nodes)
    out_x, out_n = k_context(context_W, context_nodes, neg_flat)
    return out_c, out_x, out_n


def kernel(center_nodes, context_nodes, negative_nodes, center_W, context_W):
    neg_flat = negative_nodes.reshape(_B * _NEG)
    out_c, out_x, out_n = _run(
        center_nodes.astype(jnp.int32),
        context_nodes.astype(jnp.int32),
        neg_flat.astype(jnp.int32),
        center_W,
        context_W,
    )
    return out_c, out_x, out_n.reshape(_B, _NEG, _D)


# speedup vs baseline: 1.4170x; 1.4020x over previous
import functools

import jax
import jax.numpy as jnp
from jax import lax
from jax.experimental import pallas as pl
from jax.experimental.pallas import tpu as pltpu
from jax.experimental.pallas import tpu_sc as plsc

_B = 16384
_V = 1000000
_D = 64
_NEG = 5

_NC = 2
_NS = 16
_NW = _NC * _NS

_CB = _B // _NW
_NB = _B * _NEG // _NW
_CHUNK = 256
_NBUF = 3
_INFLIGHT = 48


_LANES = 16
_GROUPS = _CHUNK // _LANES
_INFLIGHT_G = _INFLIGHT // _LANES


def _fire_chunk(table, idx_v, rows_buf, gsem):
    def body(g, carry):
        @pl.when(g < _GROUPS)
        def _():
            v = idx_v[pl.ds(g * _LANES, _LANES)]
            for l in range(_LANES):
                pltpu.make_async_copy(
                    table.at[pl.ds(v[l], 1)],
                    rows_buf.at[pl.ds(g * _LANES + l, 1)], gsem,
                ).start()

        @pl.when(g >= _INFLIGHT_G)
        def _():
            k = (g - _INFLIGHT_G) * _LANES
            for l in range(_LANES):
                pltpu.make_async_copy(
                    table.at[pl.ds(0, 1)],
                    rows_buf.at[pl.ds(k + l, 1)], gsem,
                ).wait()

        return carry

    lax.fori_loop(0, _GROUPS + _INFLIGHT_G, body, 0)


def _body(center_w, context_w, cidx, xidx, nidx, out_c, out_x, out_n,
          idx_v, rows_v, gsem, ssem):
    wid = lax.axis_index("s") * _NC + lax.axis_index("c")
    base = wid * _CB
    nbase = wid * _NB

    schedule = []
    for j in range(_CB // _CHUNK):
        o = base + j * _CHUNK
        schedule.append((cidx, o, center_w, out_c, o))
    for j in range(_CB // _CHUNK):
        o = base + j * _CHUNK
        schedule.append((xidx, o, context_w, out_x, o))
    for j in range(_NB // _CHUNK):
        o = nbase + j * _CHUNK
        schedule.append((nidx, o, context_w, out_n, o))
    n = len(schedule)

    scatters = [None] * n
    for i, (src, soff, table, out, off) in enumerate(schedule):
        if i >= _NBUF:
            scatters[i - _NBUF].wait()
        pltpu.sync_copy(src.at[pl.ds(soff, _CHUNK)], idx_v)
        _fire_chunk(table, idx_v, rows_v.at[i % _NBUF], gsem)
        scatters[i] = pltpu.async_copy(
            rows_v.at[i % _NBUF], out.at[pl.ds(off, _CHUNK)], ssem)
    for i in range(n - _NBUF, n):
        scatters[i].wait()


@jax.jit
def _run(cidx, xidx, nidx, center_W, context_W):
    mesh = plsc.VectorSubcoreMesh(core_axis_name="c", subcore_axis_name="s")
    fn = functools.partial(
        pl.kernel,
        mesh=mesh,
        out_type=(
            jax.ShapeDtypeStruct((_B, _D), jnp.float32),
            jax.ShapeDtypeStruct((_B, _D), jnp.float32),
            jax.ShapeDtypeStruct((_B * _NEG, _D), jnp.float32),
        ),
        scratch_types=[
            pltpu.VMEM((_CHUNK,), jnp.int32),
            pltpu.VMEM((_NBUF, _CHUNK, _D), jnp.float32),
            pltpu.SemaphoreType.DMA,
            pltpu.SemaphoreType.DMA,
        ],
        compiler_params=pltpu.CompilerParams(use_tc_tiling_on_sc=True),
    )(_body)
    return fn(center_W, context_W, cidx, xidx, nidx)


def kernel(center_nodes, context_nodes, negative_nodes, center_W, context_W):
    out_c, out_x, out_n = _run(
        center_nodes.astype(jnp.int32),
        context_nodes.astype(jnp.int32),
        negative_nodes.astype(jnp.int32).reshape(_B * _NEG),
        center_W,
        context_W,
    )
    return out_c, out_x, out_n.reshape(_B, _NEG, _D)


# speedup vs baseline: 1.5004x; 1.0589x over previous
import functools

import jax
import jax.numpy as jnp
from jax import lax
from jax.experimental import pallas as pl
from jax.experimental.pallas import tpu as pltpu
from jax.experimental.pallas import tpu_sc as plsc

_B = 16384
_V = 1000000
_D = 64
_NEG = 5

_NC = 2
_NS = 16
_NW = _NC * _NS

_CB = _B // _NW
_NB = _B * _NEG // _NW
_CHUNK = 256
_NBUF = 3
_INFLIGHT = 48

_LANES = 16
_GROUPS = _CHUNK // _LANES
_INFLIGHT_G = _INFLIGHT // _LANES


def _fire_chunk(table, idx_v, rows_buf, gsem):
    def body(g, carry):
        @pl.when(g < _GROUPS)
        def _():
            v = idx_v[pl.ds(g * _LANES, _LANES)]
            for l in range(_LANES):
                pltpu.make_async_copy(
                    table.at[pl.ds(v[l], 1)],
                    rows_buf.at[pl.ds(g * _LANES + l, 1)], gsem,
                ).start()

        @pl.when(g >= _INFLIGHT_G)
        def _():
            k = (g - _INFLIGHT_G) * _LANES
            for l in range(_LANES):
                pltpu.make_async_copy(
                    table.at[pl.ds(0, 1)],
                    rows_buf.at[pl.ds(k + l, 1)], gsem,
                ).wait()

        return carry

    lax.fori_loop(0, _GROUPS + _INFLIGHT_G, body, 0)


def _gather_pipeline(schedule, idx_v, rows_v, gsem, ssem):
    n = len(schedule)
    scatters = [None] * n
    for i, (src, soff, table, out, off) in enumerate(schedule):
        if i >= _NBUF:
            scatters[i - _NBUF].wait()
        pltpu.sync_copy(src.at[pl.ds(soff, _CHUNK)], idx_v)
        _fire_chunk(table, idx_v, rows_v.at[i % _NBUF], gsem)
        scatters[i] = pltpu.async_copy(
            rows_v.at[i % _NBUF], out.at[pl.ds(off, _CHUNK)], ssem)
    for i in range(max(0, n - _NBUF), n):
        scatters[i].wait()


def _center_body(center_w, cidx, out_c, idx_v, rows_v, gsem, ssem):
    wid = lax.axis_index("s") * _NC + lax.axis_index("c")
    base = wid * _CB
    schedule = [(cidx, base + j * _CHUNK, center_w, out_c, base + j * _CHUNK)
                for j in range(_CB // _CHUNK)]
    _gather_pipeline(schedule, idx_v, rows_v, gsem, ssem)


def _context_body(context_w, xidx, nidx, out_x, out_n, idx_v, rows_v,
                  gsem, ssem):
    wid = lax.axis_index("s") * _NC + lax.axis_index("c")
    base = wid * _CB
    nbase = wid * _NB
    schedule = [(xidx, base + j * _CHUNK, context_w, out_x,
                 base + j * _CHUNK) for j in range(_CB // _CHUNK)]
    schedule += [(nidx, nbase + j * _CHUNK, context_w, out_n,
                  nbase + j * _CHUNK) for j in range(_NB // _CHUNK)]
    _gather_pipeline(schedule, idx_v, rows_v, gsem, ssem)


def _make_kernel(body, out_type):
    mesh = plsc.VectorSubcoreMesh(core_axis_name="c", subcore_axis_name="s")
    return functools.partial(
        pl.kernel,
        mesh=mesh,
        out_type=out_type,
        scratch_types=[
            pltpu.VMEM((_CHUNK,), jnp.int32),
            pltpu.VMEM((_NBUF, _CHUNK, _D), jnp.float32),
            pltpu.SemaphoreType.DMA,
            pltpu.SemaphoreType.DMA,
        ],
        compiler_params=pltpu.CompilerParams(use_tc_tiling_on_sc=True),
    )(body)


@jax.jit
def _run(cidx, xidx, nidx, center_W, context_W):
    k_center = _make_kernel(
        _center_body, jax.ShapeDtypeStruct((_B, _D), jnp.float32))
    k_context = _make_kernel(
        _context_body,
        (jax.ShapeDtypeStruct((_B, _D), jnp.float32),
         jax.ShapeDtypeStruct((_B * _NEG, _D), jnp.float32)))
    out_c = k_center(center_W, cidx)
    out_x, out_n = k_context(context_W, xidx, nidx)
    return out_c, out_x, out_n


def kernel(center_nodes, context_nodes, negative_nodes, center_W, context_W):
    out_c, out_x, out_n = _run(
        center_nodes.astype(jnp.int32),
        context_nodes.astype(jnp.int32),
        negative_nodes.astype(jnp.int32).reshape(_B * _NEG),
        center_W,
        context_W,
    )
    return out_c, out_x, out_n.reshape(_B, _NEG, _D)


# speedup vs baseline: 1.5011x; 1.0005x over previous
import functools

import jax
import jax.numpy as jnp
from jax import lax
from jax.experimental import pallas as pl
from jax.experimental.pallas import tpu as pltpu
from jax.experimental.pallas import tpu_sc as plsc

_B = 16384
_V = 1000000
_D = 64
_NEG = 5

_NC = 2
_NS = 16
_NW = _NC * _NS

_CB = _B // _NW
_NB = _B * _NEG // _NW
_CHUNK = 256
_NBUF = 3
_INFLIGHT = 48

_LANES = 16
_GROUPS = _CHUNK // _LANES
_INFLIGHT_G = _INFLIGHT // _LANES


def _fire_chunk(table, idx_v, rows_buf, gsem):
    def body(g, carry):
        @pl.when(g < _GROUPS)
        def _():
            v = idx_v[pl.ds(g * _LANES, _LANES)]
            for l in range(_LANES):
                pltpu.make_async_copy(
                    table.at[pl.ds(v[l], 1)],
                    rows_buf.at[pl.ds(g * _LANES + l, 1)], gsem,
                ).start()

        @pl.when(g >= _INFLIGHT_G)
        def _():
            k = (g - _INFLIGHT_G) * _LANES
            for l in range(_LANES):
                pltpu.make_async_copy(
                    table.at[pl.ds(0, 1)],
                    rows_buf.at[pl.ds(k + l, 1)], gsem,
                ).wait()

        return carry

    lax.fori_loop(0, _GROUPS + _INFLIGHT_G, body, 0)


def _gather_pipeline(schedule, idx_v, rows_v, gsem, ssem):
    n = len(schedule)
    scatters = [None] * n
    for i, (src, soff, table, out, off) in enumerate(schedule):
        if i >= _NBUF:
            scatters[i - _NBUF].wait()
        pltpu.sync_copy(src.at[pl.ds(soff, _CHUNK)], idx_v)
        _fire_chunk(table, idx_v, rows_v.at[i % _NBUF], gsem)
        scatters[i] = pltpu.async_copy(
            rows_v.at[i % _NBUF], out.at[pl.ds(off, _CHUNK)], ssem)
    for i in range(max(0, n - _NBUF), n):
        scatters[i].wait()


def _center_body(center_w, cidx, out_c, idx_v, rows_v, gsem, ssem):
    wid = lax.axis_index("s") * _NC + lax.axis_index("c")
    base = wid * _CB
    schedule = [(cidx, base + j * _CHUNK, center_w, out_c, base + j * _CHUNK)
                for j in range(_CB // _CHUNK)]
    _gather_pipeline(schedule, idx_v, rows_v, gsem, ssem)


def _context_body(context_w, xidx, nidx, out_x, out_n, idx_v, rows_v,
                  gsem, ssem):
    wid = lax.axis_index("s") * _NC + lax.axis_index("c")
    base = wid * _CB
    nbase = wid * _NB
    schedule = [(xidx, base + j * _CHUNK, context_w, out_x,
                 base + j * _CHUNK) for j in range(_CB // _CHUNK)]
    schedule += [(nidx, nbase + j * _CHUNK, context_w, out_n,
                  nbase + j * _CHUNK) for j in range(_NB // _CHUNK)]
    _gather_pipeline(schedule, idx_v, rows_v, gsem, ssem)


def _make_kernel(body, out_type):
    mesh = plsc.VectorSubcoreMesh(core_axis_name="c", subcore_axis_name="s")
    return functools.partial(
        pl.kernel,
        mesh=mesh,
        out_type=out_type,
        scratch_types=[
            pltpu.VMEM((_CHUNK,), jnp.int32),
            pltpu.VMEM((_NBUF, _CHUNK, _D), jnp.float32),
            pltpu.SemaphoreType.DMA,
            pltpu.SemaphoreType.DMA,
        ],
        compiler_params=pltpu.CompilerParams(use_tc_tiling_on_sc=True),
    )(body)


@jax.jit
def _run(cidx, xidx, nidx, center_W, context_W):
    k_center = _make_kernel(
        _center_body, jax.ShapeDtypeStruct((_B, _D), jnp.float32))
    k_context = _make_kernel(
        _context_body,
        (jax.ShapeDtypeStruct((_B, _D), jnp.float32),
         jax.ShapeDtypeStruct((_B * _NEG, _D), jnp.float32)))
    out_x, out_n = k_context(context_W, xidx, nidx)
    out_c = k_center(center_W, cidx)
    return out_c, out_x, out_n


def kernel(center_nodes, context_nodes, negative_nodes, center_W, context_W):
    out_c, out_x, out_n = _run(
        center_nodes.astype(jnp.int32),
        context_nodes.astype(jnp.int32),
        negative_nodes.astype(jnp.int32).reshape(_B * _NEG),
        center_W,
        context_W,
    )
    return out_c, out_x, out_n.reshape(_B, _NEG, _D)


# speedup vs baseline: 1.5571x; 1.0373x over previous
import functools

import jax
import jax.numpy as jnp
from jax import lax
from jax.experimental import pallas as pl
from jax.experimental.pallas import tpu as pltpu
from jax.experimental.pallas import tpu_sc as plsc

_B = 16384
_V = 1000000
_D = 64
_NEG = 5

_NC = 2
_NS = 16
_NW = _NC * _NS

_CB = _B // _NW
_NB = _B * _NEG // _NW
_CHUNK = 256
_NBUF = 3
_INFLIGHT = 48

_LANES = 16
_GROUPS = _CHUNK // _LANES
_INFLIGHT_G = _INFLIGHT // _LANES


def _fire_chunk(table, idx_v, rows_buf, gsem):
    def body(g, carry):
        @pl.when(g < _GROUPS)
        def _():
            v = idx_v[pl.ds(g * _LANES, _LANES)]
            for l in range(_LANES):
                pltpu.make_async_copy(
                    table.at[pl.ds(v[l], 1)],
                    rows_buf.at[pl.ds(g * _LANES + l, 1)], gsem,
                ).start()

        @pl.when(g >= _INFLIGHT_G)
        def _():
            k = (g - _INFLIGHT_G) * _LANES
            for l in range(_LANES):
                pltpu.make_async_copy(
                    table.at[pl.ds(0, 1)],
                    rows_buf.at[pl.ds(k + l, 1)], gsem,
                ).wait()

        return carry

    lax.fori_loop(0, _GROUPS + _INFLIGHT_G, body, 0)


def _scatter_rows_3d(out3, rows_buf, goff, ssem):
    def body(g, carry):
        for l in range(_LANES):
            j = g * _LANES + l
            gr = goff + j
            b = gr // _NEG
            nn = gr - b * _NEG
            pltpu.make_async_copy(
                rows_buf.at[pl.ds(j, 1)], out3.at[b, pl.ds(nn, 1)], ssem
            ).start()
        return carry

    lax.fori_loop(0, _GROUPS, body, 0)


def _drain_rows_3d(out3, rows_buf, ssem):
    def body(g, carry):
        for l in range(_LANES):
            j = g * _LANES + l
            pltpu.make_async_copy(
                rows_buf.at[pl.ds(j, 1)], out3.at[0, pl.ds(0, 1)], ssem
            ).wait()
        return carry

    lax.fori_loop(0, _GROUPS, body, 0)


def _gather_pipeline(schedule, idx_v, rows_v, gsem, ssem, s3sem):
    n = len(schedule)
    scatters = [None] * n

    def _wait_chunk(i):
        _, _, _, out, _, is_3d = schedule[i]
        if is_3d:
            _drain_rows_3d(out, rows_v.at[i % _NBUF], s3sem)
        else:
            scatters[i].wait()

    for i, (src, soff, table, out, off, is_3d) in enumerate(schedule):
        if i >= _NBUF:
            _wait_chunk(i - _NBUF)
        pltpu.sync_copy(src.at[pl.ds(soff, _CHUNK)], idx_v)
        _fire_chunk(table, idx_v, rows_v.at[i % _NBUF], gsem)
        if is_3d:
            _scatter_rows_3d(out, rows_v.at[i % _NBUF], off, s3sem)
        else:
            scatters[i] = pltpu.async_copy(
                rows_v.at[i % _NBUF], out.at[pl.ds(off, _CHUNK)], ssem)
    for i in range(max(0, n - _NBUF), n):
        _wait_chunk(i)


def _center_body(center_w, cidx, out_c, idx_v, rows_v, gsem, ssem, s3sem):
    wid = lax.axis_index("s") * _NC + lax.axis_index("c")
    base = wid * _CB
    schedule = [(cidx, base + j * _CHUNK, center_w, out_c,
                 base + j * _CHUNK, False) for j in range(_CB // _CHUNK)]
    _gather_pipeline(schedule, idx_v, rows_v, gsem, ssem, s3sem)


def _context_body(context_w, xidx, nidx, out_x, out_n, idx_v, rows_v,
                  gsem, ssem, s3sem):
    wid = lax.axis_index("s") * _NC + lax.axis_index("c")
    base = wid * _CB
    nbase = wid * _NB
    schedule = [(xidx, base + j * _CHUNK, context_w, out_x,
                 base + j * _CHUNK, False) for j in range(_CB // _CHUNK)]
    schedule += [(nidx, nbase + j * _CHUNK, context_w, out_n,
                  nbase + j * _CHUNK, True) for j in range(_NB // _CHUNK)]
    _gather_pipeline(schedule, idx_v, rows_v, gsem, ssem, s3sem)


def _make_kernel(body, out_type):
    mesh = plsc.VectorSubcoreMesh(core_axis_name="c", subcore_axis_name="s")
    return functools.partial(
        pl.kernel,
        mesh=mesh,
        out_type=out_type,
        scratch_types=[
            pltpu.VMEM((_CHUNK,), jnp.int32),
            pltpu.VMEM((_NBUF, _CHUNK, _D), jnp.float32),
            pltpu.SemaphoreType.DMA,
            pltpu.SemaphoreType.DMA,
            pltpu.SemaphoreType.DMA,
        ],
        compiler_params=pltpu.CompilerParams(use_tc_tiling_on_sc=True),
    )(body)


@jax.jit
def _run(cidx, xidx, nidx, center_W, context_W):
    k_center = _make_kernel(
        _center_body, jax.ShapeDtypeStruct((_B, _D), jnp.float32))
    k_context = _make_kernel(
        _context_body,
        (jax.ShapeDtypeStruct((_B, _D), jnp.float32),
         jax.ShapeDtypeStruct((_B, _NEG, _D), jnp.float32)))
    out_x, out_n = k_context(context_W, xidx, nidx)
    out_c = k_center(center_W, cidx)
    return out_c, out_x, out_n


def kernel(center_nodes, context_nodes, negative_nodes, center_W, context_W):
    out_c, out_x, out_n = _run(
        center_nodes.astype(jnp.int32),
        context_nodes.astype(jnp.int32),
        negative_nodes.astype(jnp.int32).reshape(_B * _NEG),
        center_W,
        context_W,
    )
    return out_c, out_x, out_n
